# baseline (speedup 1.0000x reference)
import functools

import jax
import jax.numpy as jnp
from jax import lax
from jax.experimental import pallas as pl
from jax.experimental.pallas import tpu as pltpu
from jax.experimental.pallas import tpu_sc as plsc

_NUM_CLASSES = 10
_ORDERED_P = 0.5
_NT = _NUM_CLASSES - 1

_NC, _NS, _L = 2, 16, 16
_NW = _NC * _NS
_CR = 125


def _class_table(num_cols):
    key = jax.random.key(42)
    kr, kv, kp = jax.random.split(key, 3)
    randomized = jax.random.uniform(kr, (num_cols,)) > _ORDERED_P
    reverse = jax.random.uniform(kv, (num_cols,)) > 0.5
    perm = jax.random.permutation(kp, _NUM_CLASSES).astype(jnp.int32)
    c = jnp.arange(_NUM_CLASSES, dtype=jnp.int32)
    m = jnp.where(randomized[:, None], perm[None, :], c[None, :])
    m = jnp.where(reverse[:, None], _NUM_CLASSES - 1 - m, m)
    return m


def _sc_call(t_sc, b):
    n_chunks = t_sc // (_NW * _CR)
    rpw = t_sc // _NW
    ngrp = b // _L

    mesh = plsc.VectorSubcoreMesh(core_axis_name="c", subcore_axis_name="s")

    @functools.partial(
        pl.kernel,
        mesh=mesh,
        out_type=jax.ShapeDtypeStruct((t_sc * b,), jnp.int32),
        scratch_types=[
            pltpu.VMEM((_CR * b,), jnp.float32),
            pltpu.VMEM((_CR * b,), jnp.int32),
            pltpu.VMEM((_NT * _L,), jnp.float32),
            pltpu.VMEM((b // _L * _NT * _L,), jnp.int32),
            pltpu.VMEM((b,), jnp.int32),
        ],
    )
    def k(x_hbm, s_hbm, d_hbm, l0_hbm, out_hbm, x_v, o_v, s_v, d_v, l0_v):
        wid = lax.axis_index("s") * _NC + lax.axis_index("c")
        pltpu.sync_copy(s_hbm, s_v)
        pltpu.sync_copy(d_hbm, d_v)
        pltpu.sync_copy(l0_hbm, l0_v)
        row0 = wid * rpw

        def chunk_body(ci, _):
            off = (row0 + ci * _CR) * b
            pltpu.sync_copy(x_hbm.at[pl.ds(off, _CR * b)], x_v)
            for g in range(ngrp):
                l0g = l0_v[pl.ds(g * _L, _L)]
                dg = [d_v[pl.ds((g * _NT + i) * _L, _L)] for i in range(_NT)]
                ss = [s_v[pl.ds(i * _L, _L)] for i in range(_NT)]

                def row_body(r, _):
                    p = r * b + g * _L
                    xv = x_v[pl.ds(p, _L)]
                    acc = l0g
                    for i in range(_NT):
                        acc = jnp.where(xv > ss[i], acc + dg[i], acc)
                    o_v[pl.ds(p, _L)] = acc
                    return 0

                lax.fori_loop(0, _CR, row_body, 0)
            pltpu.sync_copy(o_v, out_hbm.at[pl.ds(off, _CR * b)])
            return 0

        lax.fori_loop(0, n_chunks, chunk_body, 0)

    return k


def kernel(x, thresholds):
    t, b = x.shape
    m = _class_table(b)
    s_sorted = jnp.sort(thresholds)
    d = (m[:, 1:] - m[:, :-1]).T
    l0 = m[:, 0]

    s16 = jnp.broadcast_to(s_sorted[:, None], (_NT, _L)).reshape(-1)
    d_sc = d.reshape(_NT, b // _L, _L).transpose(1, 0, 2).reshape(-1)

    out_flat = _sc_call(t, b)(x.reshape(-1), s16, d_sc, l0)
    return out_flat.reshape(t, b)


# speedup vs baseline: 1.0715x; 1.0715x over previous
import functools

import jax
import jax.numpy as jnp
from jax import lax
from jax.experimental import pallas as pl
from jax.experimental.pallas import tpu as pltpu
from jax.experimental.pallas import tpu_sc as plsc

_NUM_CLASSES = 10
_ORDERED_P = 0.5
_NT = _NUM_CLASSES - 1

_NC, _NS, _L = 2, 16, 16
_NW = _NC * _NS
_CR = 25
_NBUF = 5


def _class_table(num_cols):
    key = jax.random.key(42)
    kr, kv, kp = jax.random.split(key, 3)
    randomized = jax.random.uniform(kr, (num_cols,)) > _ORDERED_P
    reverse = jax.random.uniform(kv, (num_cols,)) > 0.5
    perm = jax.random.permutation(kp, _NUM_CLASSES).astype(jnp.int32)
    c = jnp.arange(_NUM_CLASSES, dtype=jnp.int32)
    m = jnp.where(randomized[:, None], perm[None, :], c[None, :])
    m = jnp.where(reverse[:, None], _NUM_CLASSES - 1 - m, m)
    return m


def _sc_call(t_sc, b):
    cw = _CR * b
    n_chunks = t_sc // (_NW * _CR)
    n_outer = n_chunks // _NBUF
    rpw = t_sc // _NW
    ngrp = b // _L

    mesh = plsc.VectorSubcoreMesh(core_axis_name="c", subcore_axis_name="s")

    @functools.partial(
        pl.kernel,
        mesh=mesh,
        out_type=jax.ShapeDtypeStruct((t_sc * b,), jnp.int32),
        scratch_types=[pltpu.VMEM((cw,), jnp.float32)] * _NBUF
        + [pltpu.VMEM((cw,), jnp.int32)] * _NBUF
        + [
            pltpu.VMEM((_NT * _L,), jnp.float32),
            pltpu.VMEM((ngrp * _NT * _L,), jnp.int32),
            pltpu.VMEM((b,), jnp.int32),
        ]
        + [pltpu.SemaphoreType.DMA] * (2 * _NBUF),
    )
    def k(x_hbm, s_hbm, d_hbm, l0_hbm, out_hbm, *refs):
        x_v = refs[:_NBUF]
        o_v = refs[_NBUF : 2 * _NBUF]
        s_v, d_v, l0_v = refs[2 * _NBUF : 2 * _NBUF + 3]
        sems = refs[2 * _NBUF + 3 :]
        in_sems, out_sems = sems[:_NBUF], sems[_NBUF:]
        wid = lax.axis_index("s") * _NC + lax.axis_index("c")
        pltpu.sync_copy(s_hbm, s_v)
        pltpu.sync_copy(d_hbm, d_v)
        pltpu.sync_copy(l0_hbm, l0_v)
        base = wid * rpw * b

        def in_dma(ci, slot):
            return pltpu.make_async_copy(
                x_hbm.at[pl.ds(base + ci * cw, cw)], x_v[slot], in_sems[slot]
            )

        def out_dma(ci, slot):
            return pltpu.make_async_copy(
                o_v[slot], out_hbm.at[pl.ds(base + ci * cw, cw)], out_sems[slot]
            )

        for slot in range(_NBUF):
            in_dma(slot, slot).start()

        def compute_chunk(slot):
            xs, os = x_v[slot], o_v[slot]
            for g in range(ngrp):
                l0g = l0_v[pl.ds(g * _L, _L)]
                dg = [d_v[pl.ds((g * _NT + i) * _L, _L)] for i in range(_NT)]
                ss = [s_v[pl.ds(i * _L, _L)] for i in range(_NT)]

                def row_body(r, _):
                    p = r * b + g * _L
                    xv = xs[pl.ds(p, _L)]
                    acc = l0g
                    for i in range(_NT):
                        acc = jnp.where(xv > ss[i], acc + dg[i], acc)
                    os[pl.ds(p, _L)] = acc
                    return 0

                lax.fori_loop(0, _CR, row_body, 0)

        def outer_body(j, _):
            for slot in range(_NBUF):
                ci = j * _NBUF + slot
                in_dma(ci, slot).wait()

                @pl.when(j > 0)
                def _():
                    out_dma(ci - _NBUF, slot).wait()

                compute_chunk(slot)
                out_dma(ci, slot).start()

                @pl.when(ci + _NBUF < n_chunks)
                def _():
                    in_dma(ci + _NBUF, slot).start()

            return 0

        lax.fori_loop(0, n_outer, outer_body, 0)
        for slot in range(_NBUF):
            out_dma(n_chunks - _NBUF + slot, slot).wait()

    return k


def kernel(x, thresholds):
    t, b = x.shape
    m = _class_table(b)
    s_sorted = jnp.sort(thresholds)
    d = (m[:, 1:] - m[:, :-1]).T
    l0 = m[:, 0]

    s16 = jnp.broadcast_to(s_sorted[:, None], (_NT, _L)).reshape(-1)
    d_sc = d.reshape(_NT, b // _L, _L).transpose(1, 0, 2).reshape(-1)

    out_flat = _sc_call(t, b)(x.reshape(-1), s16, d_sc, l0)
    return out_flat.reshape(t, b)
